# Initial kernel scaffold; baseline (speedup 1.0000x reference)
import jax
import jax.numpy as jnp
from jax.experimental import pallas as pl


def kernel(indices, table):
    raise NotImplementedError("write your pallas kernel here")


# speedup vs baseline: 3.6448x; 3.6448x over previous
import functools

import jax
import jax.numpy as jnp
from jax import lax
from jax.experimental import pallas as pl
from jax.experimental.pallas import tpu as pltpu
from jax.experimental.pallas import tpu_sc as plsc

_CHUNK = 128


def _make_gather(vocab: int, emb: int, nrows: int):
    info = plsc.get_sparse_core_info()
    nw = info.num_cores * info.num_subcores
    assert nrows % (nw * _CHUNK) == 0
    rows_per_w = nrows // nw
    chunks_per_w = rows_per_w // _CHUNK

    mesh = plsc.VectorSubcoreMesh(core_axis_name="c", subcore_axis_name="s")

    @functools.partial(
        pl.kernel,
        mesh=mesh,
        out_type=jax.ShapeDtypeStruct((nrows, emb), jnp.float32),
        scratch_types=[
            pltpu.VMEM((rows_per_w,), jnp.int32),
            pltpu.VMEM((_CHUNK, emb), jnp.float32),
            pltpu.SemaphoreType.DMA,
        ],
        compiler_params=pltpu.CompilerParams(use_tc_tiling_on_sc=False),
    )
    def gather_kernel(table_hbm, idx_hbm, out_hbm, idx_v, rows_v, sem):
        wid = lax.axis_index("s") * info.num_cores + lax.axis_index("c")
        base = wid * rows_per_w
        pltpu.sync_copy(idx_hbm.at[pl.ds(base, rows_per_w)], idx_v)

        def body(c):
            off = c * _CHUNK
            pltpu.async_copy(
                table_hbm.at[idx_v.at[pl.ds(off, _CHUNK)]], rows_v, sem
            ).wait()
            pltpu.sync_copy(rows_v, out_hbm.at[pl.ds(base + off, _CHUNK)])

        pl.loop(0, chunks_per_w)(body)

    return gather_kernel


def kernel(indices, table):
    batch, hist = indices.shape
    vocab, emb = table.shape
    idx_flat = indices.T.reshape(-1)
    out = _make_gather(vocab, emb, batch * hist)(table, idx_flat)
    return out.reshape(hist, batch, emb)


# speedup vs baseline: 4.3698x; 1.1989x over previous
import functools

import jax
import jax.numpy as jnp
from jax import lax
from jax.experimental import pallas as pl
from jax.experimental.pallas import tpu as pltpu
from jax.experimental.pallas import tpu_sc as plsc

_CHUNK = 128
_GRP = 4


def _make_gather(vocab: int, emb: int, nrows: int):
    info = plsc.get_sparse_core_info()
    nw = info.num_cores * info.num_subcores
    assert nrows % (nw * _CHUNK * 2 * _GRP) == 0
    rows_per_w = nrows // nw
    chunks_per_w = rows_per_w // _CHUNK
    nsuper = chunks_per_w // (2 * _GRP)
    grp_rows = _GRP * _CHUNK

    mesh = plsc.VectorSubcoreMesh(core_axis_name="c", subcore_axis_name="s")

    @functools.partial(
        pl.kernel,
        mesh=mesh,
        out_type=jax.ShapeDtypeStruct((nrows, emb), jnp.float32),
        scratch_types=[
            pltpu.VMEM((rows_per_w,), jnp.int32),
            pltpu.VMEM((grp_rows, emb), jnp.float32),
            pltpu.VMEM((grp_rows, emb), jnp.float32),
            pltpu.SemaphoreType.DMA,
            pltpu.SemaphoreType.DMA,
            pltpu.SemaphoreType.DMA,
            pltpu.SemaphoreType.DMA,
        ],
        compiler_params=pltpu.CompilerParams(use_tc_tiling_on_sc=False),
    )
    def gather_kernel(table_hbm, idx_hbm, out_hbm, idx_v, buf_a, buf_b,
                      gsem_a, gsem_b, osem_a, osem_b):
        wid = lax.axis_index("s") * info.num_cores + lax.axis_index("c")
        base = wid * rows_per_w
        pltpu.sync_copy(idx_hbm.at[pl.ds(base, rows_per_w)], idx_v)

        def fire_group(c0, buf, gsem):
            for j in range(_GRP):
                pltpu.async_copy(
                    table_hbm.at[idx_v.at[pl.ds((c0 + j) * _CHUNK, _CHUNK)]],
                    buf.at[pl.ds(j * _CHUNK, _CHUNK)],
                    gsem,
                )

        def drain_gathers(buf, gsem):
            for j in range(_GRP):
                pltpu.make_async_copy(
                    table_hbm.at[idx_v.at[pl.ds(j * _CHUNK, _CHUNK)]],
                    buf.at[pl.ds(j * _CHUNK, _CHUNK)],
                    gsem,
                ).wait()

        def fire_out(c0, buf, osem):
            pltpu.async_copy(buf, out_hbm.at[pl.ds(base + c0 * _CHUNK, grp_rows)], osem)

        def drain_out(c0, buf, osem):
            pltpu.make_async_copy(
                buf, out_hbm.at[pl.ds(base + c0 * _CHUNK, grp_rows)], osem
            ).wait()

        fire_group(0, buf_a, gsem_a)
        fire_group(_GRP, buf_b, gsem_b)

        def body(s):
            c0 = s * 2 * _GRP
            drain_gathers(buf_a, gsem_a)
            fire_out(c0, buf_a, osem_a)
            drain_gathers(buf_b, gsem_b)
            fire_out(c0 + _GRP, buf_b, osem_b)
            drain_out(c0, buf_a, osem_a)
            fire_group(c0 + 2 * _GRP, buf_a, gsem_a)
            drain_out(c0 + _GRP, buf_b, osem_b)
            fire_group(c0 + 3 * _GRP, buf_b, gsem_b)

        pl.loop(0, nsuper - 1)(body)

        c0 = (nsuper - 1) * 2 * _GRP
        drain_gathers(buf_a, gsem_a)
        fire_out(c0, buf_a, osem_a)
        drain_gathers(buf_b, gsem_b)
        fire_out(c0 + _GRP, buf_b, osem_b)
        drain_out(c0, buf_a, osem_a)
        drain_out(c0 + _GRP, buf_b, osem_b)

    return gather_kernel


def kernel(indices, table):
    batch, hist = indices.shape
    vocab, emb = table.shape
    idx_flat = indices.T.reshape(-1)
    out = _make_gather(vocab, emb, batch * hist)(table, idx_flat)
    return out.reshape(hist, batch, emb)
